# baseline (speedup 1.0000x reference)
import math

import jax
import jax.numpy as jnp
from jax.experimental import pallas as pl

_NBINS = 9
_POOL = 8
_GW = 16
_H = 512
_W = 512


def _hog_slice_kernel(x_ref, kern_ref, pool_ref, out_ref):
    img = x_ref[0].astype(jnp.bfloat16).astype(jnp.float32)

    def row_m1(a):
        return jnp.concatenate([a[1:2, :], a[:-1, :]], axis=0)

    def row_p1(a):
        return jnp.concatenate([a[1:, :], a[_H - 2:_H - 1, :]], axis=0)

    def col_m1(a):
        return jnp.concatenate([a[:, 1:2], a[:, :-1]], axis=1)

    def col_p1(a):
        return jnp.concatenate([a[:, 1:], a[:, _W - 2:_W - 1]], axis=1)

    r0 = row_m1(img)
    r2 = row_p1(img)
    s00, s02 = col_m1(r0), col_p1(r0)
    s10, s12 = col_m1(img), col_p1(img)
    s20, s22 = col_m1(r2), col_p1(r2)

    gx = s00 - s02 + 2.0 * s10 - 2.0 * s12 + s20 - s22
    gy = s00 + 2.0 * r0 + s02 - s20 - 2.0 * r2 - s22

    wnorm = jnp.sqrt(gx * gx + gy * gy) * kern_ref[...]

    phase = jnp.arctan2(gx, gy) / math.pi * _NBINS
    binf = jnp.floor(phase)
    binf = binf - _NBINS * jnp.floor(binf / _NBINS)

    pmat = pool_ref[...]

    pooled = []
    for k in range(_NBINS):
        a = jnp.where(binf == float(k), wnorm, 0.0)
        rp = jax.lax.dot_general(
            pmat, a, (((0,), (0,)), ((), ())),
            preferred_element_type=jnp.float32,
            precision=jax.lax.Precision.DEFAULT)
        pooled.append(jax.lax.dot_general(
            rp, pmat, (((1,), (0,)), ((), ())),
            preferred_element_type=jnp.float32,
            precision=jax.lax.Precision.DEFAULT))
    hist = jnp.stack(pooled, axis=0)

    denom = jnp.maximum(
        jnp.sqrt(jnp.sum(hist * hist, axis=0, keepdims=True)), 1e-12)
    hn = hist / denom

    packed = jnp.transpose(
        hn.reshape(_NBINS, 16, 4, 16, 4), (1, 3, 0, 2, 4)).reshape(256, 144)
    out_ref[0, 0] = packed


def _gauss_kern(h, w):
    n = jnp.arange(_GW, dtype=jnp.float32)
    n = (n - jnp.mean(n)) / (_GW // 2)
    k1 = jnp.exp(-0.5 * n * n)
    k2 = k1[:, None] * k1[None, :]
    k2 = k2 / jnp.sum(k2)
    return jnp.tile(k2, (h // _GW, w // _GW))


def kernel(x):
    b, c, h, w = x.shape
    hp, wp = h // _POOL, w // _POOL
    bc = b * c

    kern = _gauss_kern(h, w)
    pmat = (jnp.arange(h)[:, None] // _POOL ==
            jnp.arange(hp)[None, :]).astype(jnp.float32)

    xs = x.reshape(bc, h, w)
    out = pl.pallas_call(
        _hog_slice_kernel,
        grid=(b, c),
        in_specs=[
            pl.BlockSpec((1, h, w), lambda i, j: (i * 3 + j, 0, 0)),
            pl.BlockSpec((h, w), lambda i, j: (0, 0)),
            pl.BlockSpec((h, hp), lambda i, j: (0, 0)),
        ],
        out_specs=pl.BlockSpec((1, 1, 256, 144), lambda i, j: (i, j, 0, 0)),
        out_shape=jax.ShapeDtypeStruct((b, c, 256, 144), jnp.float32),
    )(xs, kern, pmat)
    return out.transpose(0, 2, 1, 3).reshape(b, 256, c * 144)


# speedup vs baseline: 1.0440x; 1.0440x over previous
import math

import jax
import jax.numpy as jnp
from jax.experimental import pallas as pl

_NBINS = 9
_POOL = 8
_GW = 16
_H = 512
_W = 512


def _hog_slice_kernel(x_ref, kern_ref, pool_ref, out_ref):
    img = x_ref[0].astype(jnp.bfloat16).astype(jnp.float32)

    def row_m1(a):
        return jnp.concatenate([a[1:2, :], a[:-1, :]], axis=0)

    def row_p1(a):
        return jnp.concatenate([a[1:, :], a[_H - 2:_H - 1, :]], axis=0)

    def col_m1(a):
        return jnp.concatenate([a[:, 1:2], a[:, :-1]], axis=1)

    def col_p1(a):
        return jnp.concatenate([a[:, 1:], a[:, _W - 2:_W - 1]], axis=1)

    r0 = row_m1(img)
    r2 = row_p1(img)
    s00, s02 = col_m1(r0), col_p1(r0)
    s10, s12 = col_m1(img), col_p1(img)
    s20, s22 = col_m1(r2), col_p1(r2)

    gx = s00 - s02 + 2.0 * s10 - 2.0 * s12 + s20 - s22
    gy = s00 + 2.0 * r0 + s02 - s20 - 2.0 * r2 - s22

    wnorm = jnp.sqrt(gx * gx + gy * gy) * kern_ref[...]

    neg = (gx < 0.0) | ((gx == 0.0) & (gy < 0.0))
    sy = jnp.where(neg, -gx, gx)
    sx = jnp.where(neg, -gy, gy)
    ind = []
    for k in range(1, _NBINS):
        c = math.cos(k * math.pi / _NBINS)
        s = math.sin(k * math.pi / _NBINS)
        ind.append(c * sy - s * sx >= 0.0)

    pmat = pool_ref[...]

    pooled = []
    for k in range(_NBINS):
        if k == 0:
            mask = jnp.logical_not(ind[0])
        elif k == _NBINS - 1:
            mask = ind[_NBINS - 2]
        else:
            mask = ind[k - 1] & jnp.logical_not(ind[k])
        a = jnp.where(mask, wnorm, 0.0)
        rp = jax.lax.dot_general(
            pmat, a, (((0,), (0,)), ((), ())),
            preferred_element_type=jnp.float32,
            precision=jax.lax.Precision.DEFAULT)
        pooled.append(jax.lax.dot_general(
            rp, pmat, (((1,), (0,)), ((), ())),
            preferred_element_type=jnp.float32,
            precision=jax.lax.Precision.DEFAULT))
    hist = jnp.stack(pooled, axis=0)

    denom = jnp.maximum(
        jnp.sqrt(jnp.sum(hist * hist, axis=0, keepdims=True)), 1e-12)
    out_ref[0] = hist / denom


def _gauss_kern(h, w):
    n = jnp.arange(_GW, dtype=jnp.float32)
    n = (n - jnp.mean(n)) / (_GW // 2)
    k1 = jnp.exp(-0.5 * n * n)
    k2 = k1[:, None] * k1[None, :]
    k2 = k2 / jnp.sum(k2)
    return jnp.tile(k2, (h // _GW, w // _GW))


def kernel(x):
    b, c, h, w = x.shape
    hp, wp = h // _POOL, w // _POOL
    bc = b * c

    kern = _gauss_kern(h, w)
    pmat = (jnp.arange(h)[:, None] // _POOL ==
            jnp.arange(hp)[None, :]).astype(jnp.float32)

    xs = x.reshape(bc, h, w)
    hist = pl.pallas_call(
        _hog_slice_kernel,
        grid=(bc,),
        in_specs=[
            pl.BlockSpec((1, h, w), lambda i: (i, 0, 0)),
            pl.BlockSpec((h, w), lambda i: (0, 0)),
            pl.BlockSpec((h, hp), lambda i: (0, 0)),
        ],
        out_specs=pl.BlockSpec((1, _NBINS, hp, wp), lambda i: (i, 0, 0, 0)),
        out_shape=jax.ShapeDtypeStruct((bc, _NBINS, hp, wp), jnp.float32),
    )(xs, kern, pmat)

    out = hist.reshape(b, c * _NBINS, hp, wp)
    u = wp // 16
    out = jnp.transpose(out, (0, 2, 3, 1))
    out = out.reshape(b, hp // u, u, wp // u, u, c * _NBINS)
    out = jnp.transpose(out, (0, 1, 3, 5, 2, 4))
    return out.reshape(b, (hp // u) * (wp // u), c * _NBINS * u * u)


# speedup vs baseline: 1.0888x; 1.0430x over previous
import math

import jax
import jax.numpy as jnp
from jax.experimental import pallas as pl

_NBINS = 9
_POOL = 8
_GW = 16
_H = 512
_W = 512


def _hog_one_channel(img, kern, pmat):
    img = img.astype(jnp.bfloat16).astype(jnp.float32)

    def row_m1(a):
        return jnp.concatenate([a[1:2, :], a[:-1, :]], axis=0)

    def row_p1(a):
        return jnp.concatenate([a[1:, :], a[_H - 2:_H - 1, :]], axis=0)

    def col_m1(a):
        return jnp.concatenate([a[:, 1:2], a[:, :-1]], axis=1)

    def col_p1(a):
        return jnp.concatenate([a[:, 1:], a[:, _W - 2:_W - 1]], axis=1)

    r0 = row_m1(img)
    r2 = row_p1(img)
    s00, s02 = col_m1(r0), col_p1(r0)
    s10, s12 = col_m1(img), col_p1(img)
    s20, s22 = col_m1(r2), col_p1(r2)

    gx = s00 - s02 + 2.0 * s10 - 2.0 * s12 + s20 - s22
    gy = s00 + 2.0 * r0 + s02 - s20 - 2.0 * r2 - s22

    wnorm = jnp.sqrt(gx * gx + gy * gy) * kern

    neg = (gx < 0.0) | ((gx == 0.0) & (gy < 0.0))
    sy = jnp.where(neg, -gx, gx)
    sx = jnp.where(neg, -gy, gy)
    ind = []
    for k in range(1, _NBINS):
        c = math.cos(k * math.pi / _NBINS)
        s = math.sin(k * math.pi / _NBINS)
        ind.append(c * sy - s * sx >= 0.0)

    pooled = []
    for k in range(_NBINS):
        if k == 0:
            mask = jnp.logical_not(ind[0])
        elif k == _NBINS - 1:
            mask = ind[_NBINS - 2]
        else:
            mask = ind[k - 1] & jnp.logical_not(ind[k])
        a = jnp.where(mask, wnorm, 0.0)
        rp = jax.lax.dot_general(
            pmat, a, (((0,), (0,)), ((), ())),
            preferred_element_type=jnp.float32,
            precision=jax.lax.Precision.DEFAULT)
        pooled.append(jax.lax.dot_general(
            rp, pmat, (((1,), (0,)), ((), ())),
            preferred_element_type=jnp.float32,
            precision=jax.lax.Precision.DEFAULT))
    hist = jnp.stack(pooled, axis=0)

    denom = jnp.maximum(
        jnp.sqrt(jnp.sum(hist * hist, axis=0, keepdims=True)), 1e-12)
    return hist / denom


def _hog_batch_kernel(x_ref, kern_ref, pool_ref, out_ref):
    kern = kern_ref[...]
    pmat = pool_ref[...]
    for ch in range(3):
        out_ref[0, ch] = _hog_one_channel(x_ref[0, ch], kern, pmat)


def _gauss_kern(h, w):
    n = jnp.arange(_GW, dtype=jnp.float32)
    n = (n - jnp.mean(n)) / (_GW // 2)
    k1 = jnp.exp(-0.5 * n * n)
    k2 = k1[:, None] * k1[None, :]
    k2 = k2 / jnp.sum(k2)
    return jnp.tile(k2, (h // _GW, w // _GW))


def kernel(x):
    b, c, h, w = x.shape
    hp, wp = h // _POOL, w // _POOL

    kern = _gauss_kern(h, w)
    pmat = (jnp.arange(h)[:, None] // _POOL ==
            jnp.arange(hp)[None, :]).astype(jnp.float32)

    hist = pl.pallas_call(
        _hog_batch_kernel,
        grid=(b,),
        in_specs=[
            pl.BlockSpec((1, c, h, w), lambda i: (i, 0, 0, 0)),
            pl.BlockSpec((h, w), lambda i: (0, 0)),
            pl.BlockSpec((h, hp), lambda i: (0, 0)),
        ],
        out_specs=pl.BlockSpec(
            (1, c, _NBINS, hp, wp), lambda i: (i, 0, 0, 0, 0)),
        out_shape=jax.ShapeDtypeStruct((b, c, _NBINS, hp, wp), jnp.float32),
    )(x, kern, pmat)

    out = hist.reshape(b, c * _NBINS, hp, wp)
    u = wp // 16
    out = jnp.transpose(out, (0, 2, 3, 1))
    out = out.reshape(b, hp // u, u, wp // u, u, c * _NBINS)
    out = jnp.transpose(out, (0, 1, 3, 5, 2, 4))
    return out.reshape(b, (hp // u) * (wp // u), c * _NBINS * u * u)
